# Initial kernel scaffold; baseline (speedup 1.0000x reference)
import jax
import jax.numpy as jnp
from jax.experimental import pallas as pl


def kernel(x_ingredient, x_taste, edge_index, W_ing, b_ing, W_taste, b_taste, lin_src, lin_dst, k_W, k_b, q):
    raise NotImplementedError("write your pallas kernel here")


# speedup vs baseline: 19.2681x; 19.2681x over previous
import functools

import jax
import jax.numpy as jnp
from jax import lax
from jax.experimental import pallas as pl
from jax.experimental.pallas import tpu as pltpu
from jax.experimental.pallas import tpu_sc as plsc

F = 128
N = 10000
E = 320000
NEG_SLOPE = 0.2
R_ALPHA = 0.5

NC, NS, LANES = 2, 16, 16
NW = NC * NS
CHUNK = 128
NCH = -(-E // (NW * CHUNK))
EPW = NCH * CHUNK
EPAD = NW * EPW
ZR = 624
GRP = CHUNK // LANES


def _front_body(xi_ref, xt_ref, wit_ref, bi_ref, wtt_ref, bt_ref, ls_ref,
                ld_ref, h_ref, as_ref, ad_ref):
    h = jnp.dot(xi_ref[...], wit_ref[...],
                preferred_element_type=jnp.float32) + bi_ref[...][None, :]
    h_ref[...] = h
    as_ref[...] = jnp.dot(h, ls_ref[...][:, None],
                          preferred_element_type=jnp.float32)
    v = jnp.dot(wtt_ref[...], ld_ref[...][:, None],
                preferred_element_type=jnp.float32)
    cst = jnp.sum(bt_ref[...] * ld_ref[...])
    ad_ref[...] = jnp.dot(xt_ref[...], v,
                          preferred_element_type=jnp.float32) + cst


def _front(x_ing, x_taste, WiT, b_ing, WtT, b_taste, lin_src, lin_dst):
    return pl.pallas_call(
        _front_body,
        out_shape=[
            jax.ShapeDtypeStruct((N, F), jnp.float32),
            jax.ShapeDtypeStruct((N, 1), jnp.float32),
            jax.ShapeDtypeStruct((N, 1), jnp.float32),
        ],
    )(x_ing, x_taste, WiT, b_ing, WtT, b_taste, lin_src, lin_dst)


def _sc_body(h_hbm, asrc_hbm, adst_hbm, edges_hbm,
             acc_hbm, den_hbm,
             asrc_v, adst_v, idx_c, ex_v, rows_v, zden_v,
             acc_sh, den_sh):
    c = lax.axis_index("c")
    s = lax.axis_index("s")
    wid = c * NS + s
    row0 = s * ZR

    pltpu.sync_copy(asrc_hbm, asrc_v)
    pltpu.sync_copy(adst_hbm, adst_v)

    zeros16 = jnp.zeros((LANES,), jnp.float32)

    def zrow_step(i, carry):
        for j in range(F // LANES):
            rows_v[i, pl.ds(j * LANES, LANES)] = zeros16
        return carry

    lax.fori_loop(0, CHUNK, zrow_step, 0)

    def zden_step(i, carry):
        zden_v[pl.ds(i * LANES, LANES)] = zeros16
        return carry

    lax.fori_loop(0, ZR // LANES, zden_step, 0)

    for k in range(ZR // CHUNK):
        pltpu.sync_copy(rows_v, acc_sh.at[pl.ds(row0 + k * CHUNK, CHUNK)])
    pltpu.sync_copy(rows_v.at[pl.ds(0, ZR - (ZR // CHUNK) * CHUNK)],
                    acc_sh.at[pl.ds(row0 + (ZR // CHUNK) * CHUNK,
                                    ZR - (ZR // CHUNK) * CHUNK)])
    pltpu.sync_copy(zden_v, den_sh.at[pl.ds(row0, ZR)])

    @pl.when(s == 0)
    def _():
        pltpu.sync_copy(rows_v.at[pl.ds(0, 16)], acc_sh.at[pl.ds(NS * ZR, 16)])
        pltpu.sync_copy(zden_v.at[pl.ds(0, 16)], den_sh.at[pl.ds(NS * ZR, 16)])

    plsc.subcore_barrier()

    ebase = wid * EPW
    lane_iota = lax.iota(jnp.int32, LANES)

    def chunk_step(g, carry):
        pltpu.sync_copy(edges_hbm.at[wid, g], idx_c)
        pltpu.sync_copy(h_hbm.at[idx_c.at[0]], rows_v)
        for t in range(GRP):
            si = idx_c[0, pl.ds(t * LANES, LANES)]
            di = idx_c[1, pl.ds(t * LANES, LANES)]
            al = (plsc.load_gather(asrc_v, [si])
                  + plsc.load_gather(adst_v, [di]))
            al = jnp.where(al > 0, al, NEG_SLOPE * al)
            ex = jnp.exp(al)
            pos = ebase + g * CHUNK + t * LANES + lane_iota
            ex = jnp.where(pos < E, ex, 0.0)
            ex_v[pl.ds(t * LANES, LANES)] = ex
            for j in range(LANES):
                w = jnp.broadcast_to(ex[j], (LANES,))
                e = t * LANES + j
                for v in range(F // LANES):
                    sl = pl.ds(v * LANES, LANES)
                    rows_v[e, sl] = rows_v[e, sl] * w
        pltpu.sync_copy(rows_v, acc_sh.at[idx_c.at[1]], add=True)
        pltpu.sync_copy(ex_v, den_sh.at[idx_c.at[1]], add=True)
        return carry

    lax.fori_loop(0, NCH, chunk_step, 0)

    plsc.subcore_barrier()

    pltpu.sync_copy(acc_sh.at[pl.ds(row0, ZR)], acc_hbm.at[c, pl.ds(row0, ZR)])
    pltpu.sync_copy(den_sh.at[pl.ds(row0, ZR)], zden_v)
    pltpu.sync_copy(zden_v, den_hbm.at[pl.ds(c * N + row0, ZR)])

    @pl.when(s == 0)
    def _():
        pltpu.sync_copy(acc_sh.at[pl.ds(NS * ZR, 16)],
                        acc_hbm.at[c, pl.ds(NS * ZR, 16)])
        pltpu.sync_copy(den_sh.at[pl.ds(NS * ZR, 16)], zden_v.at[pl.ds(0, 16)])
        pltpu.sync_copy(zden_v.at[pl.ds(0, 16)],
                        den_hbm.at[pl.ds(c * N + NS * ZR, 16)])


def _sc_edges(h_src, a_src, a_dst, edges_pad):
    mesh = plsc.VectorSubcoreMesh(core_axis_name="c", subcore_axis_name="s",
                                  num_cores=NC, num_subcores=NS)
    f = pl.kernel(
        _sc_body,
        out_type=[
            jax.ShapeDtypeStruct((NC, N, F), jnp.float32),
            jax.ShapeDtypeStruct((NC * N,), jnp.float32),
        ],
        mesh=mesh,
        compiler_params=pltpu.CompilerParams(needs_layout_passes=False),
        scratch_types=[
            pltpu.VMEM((N,), jnp.float32),
            pltpu.VMEM((N,), jnp.float32),
            pltpu.VMEM((2, CHUNK), jnp.int32),
            pltpu.VMEM((CHUNK,), jnp.float32),
            pltpu.VMEM((CHUNK, F), jnp.float32),
            pltpu.VMEM((ZR,), jnp.float32),
            pltpu.VMEM_SHARED((N, F), jnp.float32),
            pltpu.VMEM_SHARED((N,), jnp.float32),
        ],
    )
    return f(h_src, a_src, a_dst, edges_pad)


def _final_body(acc_ref, den_ref, xt_ref, out_ref):
    acc = acc_ref[0] + acc_ref[1]
    den = den_ref[0] + den_ref[1]
    o = acc / (den + 1e-16)[:, None]
    o = jnp.maximum(o, 0.0)
    out_ref[...] = o * (1.0 - R_ALPHA) + xt_ref[...] * R_ALPHA


def _final(acc, den, x_taste):
    return pl.pallas_call(
        _final_body,
        out_shape=jax.ShapeDtypeStruct((N, F), jnp.float32),
    )(acc, den, x_taste)


@jax.jit
def kernel(x_ingredient, x_taste, edge_index, W_ing, b_ing, W_taste, b_taste,
           lin_src, lin_dst, k_W, k_b, q):
    h_src, a_src, a_dst = _front(x_ingredient, x_taste, W_ing.T, b_ing,
                                 W_taste.T, b_taste, lin_src, lin_dst)
    a_src = a_src.reshape(N)
    a_dst = a_dst.reshape(N)
    edges_pad = jnp.pad(edge_index, ((0, 0), (0, EPAD - E)))
    edges_pad = edges_pad.reshape(2, NW, NCH, CHUNK).transpose(1, 2, 0, 3)
    acc, den = _sc_edges(h_src, a_src, a_dst, edges_pad)
    out_taste = _final(acc, den.reshape(NC, N), x_taste)
    return x_ingredient, out_taste
